# baseline (speedup 1.0000x reference)
import functools

import jax
import jax.numpy as jnp
from jax import lax
from jax.experimental import pallas as pl
from jax.experimental.pallas import tpu as pltpu
from jax.experimental.pallas import tpu_sc as plsc

_PRE_SEQ_LEN = 128
_EMB = 18432
_B = 64 * 128
_NC, _NS = 2, 16
_NW = _NC * _NS
_BPW = _B // _NW
_K = 2
_NBUF = 2
_STEPS = _BPW // _K


def _sc_body(table_hbm, idx_hbm, out_hbm, idx_v, rows_v, *sems):
    gsems = sems[:_NBUF]
    ssems = sems[_NBUF:]
    wid = lax.axis_index("s") * _NC + lax.axis_index("c")

    pltpu.sync_copy(idx_hbm.at[wid], idx_v)

    def gather(buf, chunk):
        return pltpu.make_async_copy(
            table_hbm.at[idx_v.at[chunk]], rows_v.at[buf], gsems[buf])

    def scatter(buf, chunk):
        return pltpu.make_async_copy(
            rows_v.at[buf], out_hbm.at[wid].at[chunk], ssems[buf])

    for b in range(_NBUF):
        gather(b, b).start()

    def outer(i, _):
        base = i * _NBUF
        for b in range(_NBUF):
            gather(b, base + b).wait()
            scatter(b, base + b).start()
        for b in range(_NBUF):
            scatter(b, base + b).wait()
            gather(b, base + _NBUF + b).start()
        return 0

    lax.fori_loop(0, _STEPS // _NBUF - 1, outer, 0)

    last = _STEPS - _NBUF
    for b in range(_NBUF):
        gather(b, last + b).wait()
        scatter(b, last + b).start()
    for b in range(_NBUF):
        scatter(b, last + b).wait()


@jax.jit
def _sc_gather(table, idx):
    mesh = plsc.VectorSubcoreMesh(core_axis_name="c", subcore_axis_name="s")
    f = pl.kernel(
        _sc_body,
        out_type=jax.ShapeDtypeStruct((_NW, _STEPS, _K, _EMB), jnp.float32),
        mesh=mesh,
        scratch_types=[
            pltpu.VMEM((_STEPS, _K), jnp.int32),
            pltpu.VMEM((_NBUF, _K, _EMB), jnp.float32),
        ] + [pltpu.SemaphoreType.DMA] * (2 * _NBUF),
    )
    return f(table, idx)


def kernel(prefix, embedding):
    idx = prefix.reshape(_NW, _STEPS, _K)
    out = _sc_gather(embedding, idx)
    return out.reshape(64, _PRE_SEQ_LEN, _EMB)


# speedup vs baseline: 6.2945x; 6.2945x over previous
import functools

import jax
import jax.numpy as jnp
from jax import lax
from jax.experimental import pallas as pl
from jax.experimental.pallas import tpu as pltpu
from jax.experimental.pallas import tpu_sc as plsc

_PRE_SEQ_LEN = 128
_EMB = 18432
_B = 64 * 128
_NC, _NS = 2, 16
_COLS = _EMB // _NC
_RPW = _B // _NS
_K = 16
_NBUF = 2
_STEPS = _RPW // _K
_TROWS = _PRE_SEQ_LEN // _NS


def _sc_body(table_hbm, idx_hbm, out_hbm, table_s, idx_v, *sems):
    c = lax.axis_index("c")
    s = lax.axis_index("s")
    col0 = c * _COLS
    row0 = s * _RPW
    lanes = lax.iota(jnp.int32, 16)

    pltpu.sync_copy(
        table_hbm.at[pl.ds(s * _TROWS, _TROWS), pl.ds(col0, _COLS)],
        table_s.at[pl.ds(s * _TROWS, _TROWS)])
    pltpu.sync_copy(idx_hbm.at[pl.ds(row0, _RPW)], idx_v)
    plsc.subcore_barrier()

    def row_copy(row, t, buf):
        return pltpu.make_async_copy(
            table_s.at[pl.ds(t, 1)],
            out_hbm.at[pl.ds(row0 + row, 1), pl.ds(col0, _COLS)],
            sems[buf])

    def start_flight(g, buf):
        vec = idx_v[pl.ds(g * _K, _K)]
        for j in range(_K):
            row_copy(g * _K + j, vec[j], buf).start()

    def wait_flight(g, buf):
        vec = idx_v[pl.ds(g * _K, _K)]
        for j in range(_K):
            row_copy(g * _K + j, vec[j], buf).wait()

    for b in range(_NBUF):
        start_flight(b, b)

    def outer(i, _):
        base = i * _NBUF
        for b in range(_NBUF):
            wait_flight(base + b, b)
            start_flight(base + _NBUF + b, b)
        return 0

    lax.fori_loop(0, _STEPS // _NBUF - 1, outer, 0)

    last = _STEPS - _NBUF
    for b in range(_NBUF):
        wait_flight(last + b, b)


@jax.jit
def _sc_gather(table, idx):
    mesh = plsc.VectorSubcoreMesh(core_axis_name="c", subcore_axis_name="s")
    f = pl.kernel(
        _sc_body,
        out_type=jax.ShapeDtypeStruct((_B, _EMB), jnp.float32),
        mesh=mesh,
        scratch_types=[
            pltpu.VMEM_SHARED((_PRE_SEQ_LEN, _COLS), jnp.float32),
            pltpu.VMEM((_RPW,), jnp.int32),
        ] + [pltpu.SemaphoreType.DMA] * _NBUF,
    )
    return f(table, idx)


def kernel(prefix, embedding):
    idx = prefix.reshape(_B)
    out = _sc_gather(embedding, idx)
    return out.reshape(64, _PRE_SEQ_LEN, _EMB)


# speedup vs baseline: 11.0421x; 1.7542x over previous
import jax
import jax.numpy as jnp
from jax.experimental import pallas as pl
from jax.experimental.pallas import tpu as pltpu

_P = 128
_EMB = 18432


def _tc_body(idx_ref, table_ref, out_ref):
    idx = idx_ref[0, 0, :]
    iota = jax.lax.broadcasted_iota(jnp.int32, (_P, _P), 0)
    onehot = (idx[None, :] == iota).astype(jnp.float32)
    out_ref[0] = jax.lax.dot_general(
        onehot, table_ref[...],
        dimension_numbers=(((0,), (0,)), ((), ())),
        preferred_element_type=jnp.float32)


@jax.jit
def _tc_gather(table, idx3):
    return pl.pallas_call(
        _tc_body,
        grid=(64,),
        in_specs=[
            pl.BlockSpec((1, 1, _P), lambda b: (b, 0, 0)),
            pl.BlockSpec((_P, _EMB), lambda b: (0, 0)),
        ],
        out_specs=pl.BlockSpec((1, _P, _EMB), lambda b: (b, 0, 0)),
        out_shape=jax.ShapeDtypeStruct((64, _P, _EMB), jnp.float32),
    )(idx3, table)


def kernel(prefix, embedding):
    idx3 = prefix.reshape(64, 1, _P)
    return _tc_gather(embedding, idx3)
